# baseline (speedup 1.0000x reference)
import functools

import jax
import jax.numpy as jnp
from jax import lax
from jax.experimental import pallas as pl
from jax.experimental.pallas import tpu as pltpu
from jax.experimental.pallas import tpu_sc as plsc

_B, _HW, _C = 32, 1024, 768
_CHUNK = 128
_NCH = _HW // _CHUNK


def _sc_broadcast(table):
    mesh = plsc.VectorSubcoreMesh(core_axis_name="c", subcore_axis_name="s")
    info = plsc.get_sparse_core_info()
    num_cores = info.num_cores

    @functools.partial(
        pl.kernel,
        mesh=mesh,
        out_type=jax.ShapeDtypeStruct((_B, _HW, _C), jnp.float32),
        scratch_types=[pltpu.VMEM((_CHUNK, _C), jnp.float32)],
    )
    def k(table_hbm, out_hbm, buf):
        wid = lax.axis_index("s") * num_cores + lax.axis_index("c")
        for j in range(_NCH):
            pltpu.sync_copy(table_hbm.at[pl.ds(j * _CHUNK, _CHUNK)], buf)
            pltpu.sync_copy(buf, out_hbm.at[wid, pl.ds(j * _CHUNK, _CHUNK)])

    return k(table)


def kernel(inputs, table):
    del inputs
    return _sc_broadcast(table)


# speedup vs baseline: 1.6559x; 1.6559x over previous
import functools

import jax
import jax.numpy as jnp
from jax import lax
from jax.experimental import pallas as pl
from jax.experimental.pallas import tpu as pltpu
from jax.experimental.pallas import tpu_sc as plsc

_B, _HW, _C = 32, 1024, 768


def _sc_broadcast(table):
    mesh = plsc.VectorSubcoreMesh(core_axis_name="c", subcore_axis_name="s")
    info = plsc.get_sparse_core_info()
    num_cores = info.num_cores
    num_subcores = info.num_subcores
    rows_per_sub = _HW // num_subcores

    @functools.partial(
        pl.kernel,
        mesh=mesh,
        out_type=jax.ShapeDtypeStruct((_B, _HW, _C), jnp.float32),
        scratch_types=[pltpu.VMEM_SHARED((_HW, _C), jnp.float32)],
    )
    def k(table_hbm, out_hbm, shared):
        sid = lax.axis_index("s")
        wid = sid * num_cores + lax.axis_index("c")
        row0 = sid * rows_per_sub
        pltpu.sync_copy(
            table_hbm.at[pl.ds(row0, rows_per_sub)],
            shared.at[pl.ds(row0, rows_per_sub)],
        )
        plsc.subcore_barrier()
        pltpu.sync_copy(shared, out_hbm.at[wid])

    return k(table)


def kernel(inputs, table):
    del inputs
    return _sc_broadcast(table)


# speedup vs baseline: 1.6605x; 1.0028x over previous
import functools

import jax
import jax.numpy as jnp
from jax import lax
from jax.experimental import pallas as pl
from jax.experimental.pallas import tpu as pltpu
from jax.experimental.pallas import tpu_sc as plsc

_B, _HW, _C = 32, 1024, 768


def _sc_broadcast(table):
    mesh = plsc.VectorSubcoreMesh(core_axis_name="c", subcore_axis_name="s")
    info = plsc.get_sparse_core_info()
    num_cores = info.num_cores
    num_subcores = info.num_subcores
    rows_per_sub = _HW // num_subcores

    n_wr = 8
    wr_rows = _HW // n_wr

    @functools.partial(
        pl.kernel,
        mesh=mesh,
        out_type=jax.ShapeDtypeStruct((_B, _HW, _C), jnp.float32),
        scratch_types=[
            pltpu.VMEM_SHARED((_HW, _C), jnp.float32),
            pltpu.SemaphoreType.DMA,
        ],
    )
    def k(table_hbm, out_hbm, shared, sem):
        sid = lax.axis_index("s")
        wid = sid * num_cores + lax.axis_index("c")
        row0 = sid * rows_per_sub
        pltpu.sync_copy(
            table_hbm.at[pl.ds(row0, rows_per_sub)],
            shared.at[pl.ds(row0, rows_per_sub)],
        )
        plsc.subcore_barrier()
        copies = [
            pltpu.async_copy(
                shared.at[pl.ds(j * wr_rows, wr_rows)],
                out_hbm.at[wid, pl.ds(j * wr_rows, wr_rows)],
                sem,
            )
            for j in range(n_wr)
        ]
        for c in copies:
            c.wait()

    return k(table)


def kernel(inputs, table):
    del inputs
    return _sc_broadcast(table)
